# Initial kernel scaffold; baseline (speedup 1.0000x reference)
import jax
import jax.numpy as jnp
from jax.experimental import pallas as pl


def kernel(inputs, targets, momentum, features):
    raise NotImplementedError("write your pallas kernel here")


# speedup vs baseline: 2.8919x; 2.8919x over previous
import jax
import jax.numpy as jnp
from jax.experimental import pallas as pl
from jax.experimental.pallas import tpu as pltpu

_TEMP = 0.05
_B = 1024
_D = 128
_N = 100000
_BLK = 2000


def _body(x_ref, t_ref, f_ref, loss_ref, xn_ref, m_ref, s_ref, tl_ref):
    k = pl.program_id(0)
    nk = pl.num_programs(0)

    @pl.when(k == 0)
    def _():
        x = x_ref[...]
        nrm = jnp.sqrt(jnp.sum(x * x, axis=1, keepdims=True))
        xn_ref[...] = x / jnp.maximum(nrm, 1e-12)
        m_ref[...] = jnp.full((_B, 1), -1e30, jnp.float32)
        s_ref[...] = jnp.zeros((_B, 1), jnp.float32)
        tl_ref[...] = jnp.zeros((_B, 1), jnp.float32)

    xn = xn_ref[...]
    logits = jax.lax.dot_general(
        xn, f_ref[...], (((1,), (1,)), ((), ())),
        preferred_element_type=jnp.float32) * (1.0 / _TEMP)

    bm = jnp.max(logits, axis=1, keepdims=True)
    m_old = m_ref[...]
    m_new = jnp.maximum(m_old, bm)
    e = jnp.exp(logits - m_new)
    s_ref[...] = s_ref[...] * jnp.exp(m_old - m_new) + jnp.sum(
        e, axis=1, keepdims=True)
    m_ref[...] = m_new

    col = jax.lax.broadcasted_iota(jnp.int32, (_B, _BLK), 1) + k * _BLK
    hit = col == t_ref[...]
    tl_ref[...] += jnp.sum(jnp.where(hit, logits, 0.0), axis=1, keepdims=True)

    @pl.when(k == nk - 1)
    def _():
        lse = m_ref[...] + jnp.log(s_ref[...])
        loss_ref[...] = jnp.mean(lse - tl_ref[...]).reshape(1, 1)


def kernel(inputs, targets, momentum, features):
    del momentum
    t2d = targets.astype(jnp.int32).reshape(_B, 1)
    loss = pl.pallas_call(
        _body,
        grid=(_N // _BLK,),
        in_specs=[
            pl.BlockSpec((_B, _D), lambda k: (0, 0)),
            pl.BlockSpec((_B, 1), lambda k: (0, 0)),
            pl.BlockSpec((_BLK, _D), lambda k: (k, 0)),
        ],
        out_specs=pl.BlockSpec((1, 1), lambda k: (0, 0)),
        out_shape=jax.ShapeDtypeStruct((1, 1), jnp.float32),
        scratch_shapes=[
            pltpu.VMEM((_B, _D), jnp.float32),
            pltpu.VMEM((_B, 1), jnp.float32),
            pltpu.VMEM((_B, 1), jnp.float32),
            pltpu.VMEM((_B, 1), jnp.float32),
        ],
    )(inputs, t2d, features)
    return loss[0, 0]


# speedup vs baseline: 5.6483x; 1.9531x over previous
import jax
import jax.numpy as jnp
from jax.experimental import pallas as pl
from jax.experimental.pallas import tpu as pltpu
from jax.experimental.pallas import tpu_sc as plsc

_TEMP = 0.05
_SHIFT = 1.0 / _TEMP
_B = 1024
_D = 128
_N = 100000
_BLK = 2000
_GW = 128


def _stream_body(x_ref, f_ref, s_out_ref, xn_out_ref, xb_ref, s_ref):
    k = pl.program_id(0)
    nk = pl.num_programs(0)

    @pl.when(k == 0)
    def _():
        x = x_ref[...]
        nrm = jnp.sqrt(jnp.sum(x * x, axis=1, keepdims=True))
        xn = x / jnp.maximum(nrm, 1e-12)
        xn_out_ref[...] = xn
        xb_ref[...] = (xn * _SHIFT).astype(jnp.bfloat16)
        s_ref[...] = jnp.zeros((_B, 1), jnp.float32)

    logits = jax.lax.dot_general(
        xb_ref[...], f_ref[...].astype(jnp.bfloat16),
        (((1,), (1,)), ((), ())),
        preferred_element_type=jnp.float32)
    s_ref[...] += jnp.sum(jnp.exp(logits - _SHIFT), axis=1, keepdims=True)

    @pl.when(k == nk - 1)
    def _():
        s_out_ref[...] = s_ref[...]


def _combine_body(s_ref, xn_ref, g_ref, loss_ref):
    tgt = jnp.sum(xn_ref[...] * g_ref[...], axis=1, keepdims=True) * _SHIFT
    lse = _SHIFT + jnp.log(s_ref[...])
    loss_ref[...] = jnp.mean(lse - tgt).reshape(1, 1)


def _sc_gather(features, t2d):
    mesh = plsc.VectorSubcoreMesh(core_axis_name="core",
                                  subcore_axis_name="subcore")

    @pl.kernel(out_type=jax.ShapeDtypeStruct((_B, _D), jnp.float32),
               mesh=mesh)
    def gather_kernel(f_hbm, i_hbm, o_hbm):
        def body(i_vmem, o_vmem):
            pltpu.sync_copy(f_hbm.at[i_vmem.at[0]], o_vmem)

        pltpu.emit_pipeline(
            body,
            grid=(_B // _GW,),
            in_specs=[pl.BlockSpec((1, _GW), index_map=lambda i: (0, i))],
            out_specs=[pl.BlockSpec((_GW, _D), index_map=lambda i: (i, 0))],
            core_axis_name=("core", "subcore"),
            dimension_semantics=(pltpu.PARALLEL,),
        )(i_hbm, o_hbm)

    return gather_kernel(features, t2d)


def kernel(inputs, targets, momentum, features):
    del momentum
    t2d = targets.astype(jnp.int32).reshape(1, _B)
    gathered = _sc_gather(features, t2d)

    s, xn = pl.pallas_call(
        _stream_body,
        grid=(_N // _BLK,),
        in_specs=[
            pl.BlockSpec((_B, _D), lambda k: (0, 0)),
            pl.BlockSpec((_BLK, _D), lambda k: (k, 0)),
        ],
        out_specs=[
            pl.BlockSpec((_B, 1), lambda k: (0, 0)),
            pl.BlockSpec((_B, _D), lambda k: (0, 0)),
        ],
        out_shape=[
            jax.ShapeDtypeStruct((_B, 1), jnp.float32),
            jax.ShapeDtypeStruct((_B, _D), jnp.float32),
        ],
        scratch_shapes=[
            pltpu.VMEM((_B, _D), jnp.bfloat16),
            pltpu.VMEM((_B, 1), jnp.float32),
        ],
    )(inputs, features)

    loss = pl.pallas_call(
        _combine_body,
        out_shape=jax.ShapeDtypeStruct((1, 1), jnp.float32),
    )(s, xn, gathered)
    return loss[0, 0]


# speedup vs baseline: 6.2859x; 1.1129x over previous
import jax
import jax.numpy as jnp
from jax.experimental import pallas as pl
from jax.experimental.pallas import tpu as pltpu
from jax.experimental.pallas import tpu_sc as plsc

_TEMP = 0.05
_SHIFT = 1.0 / _TEMP
_B = 1024
_D = 128
_N = 100000
_BLK = 4000
_GW = 128


def _stream_body(x_ref, f_ref, s_out_ref, xn_out_ref, xb_ref, s_ref):
    k = pl.program_id(0)
    nk = pl.num_programs(0)

    @pl.when(k == 0)
    def _():
        x = x_ref[...]
        nrm = jnp.sqrt(jnp.sum(x * x, axis=1, keepdims=True))
        xn = x / jnp.maximum(nrm, 1e-12)
        xn_out_ref[...] = xn
        xb_ref[...] = (xn * _SHIFT).astype(jnp.bfloat16)
        s_ref[...] = jnp.zeros((_B, 1), jnp.float32)

    logits = jax.lax.dot_general(
        xb_ref[...], f_ref[...].astype(jnp.bfloat16),
        (((1,), (1,)), ((), ())),
        preferred_element_type=jnp.float32)
    s_ref[...] += jnp.sum(jnp.exp(logits), axis=1, keepdims=True)

    @pl.when(k == nk - 1)
    def _():
        s_out_ref[...] = s_ref[...]


def _combine_body(s_ref, xn_ref, g_ref, loss_ref):
    tgt = jnp.sum(xn_ref[...] * g_ref[...], axis=1, keepdims=True) * _SHIFT
    lse = jnp.log(s_ref[...])
    loss_ref[...] = jnp.mean(lse - tgt).reshape(1, 1)


def _sc_gather(features, t2d):
    mesh = plsc.VectorSubcoreMesh(core_axis_name="core",
                                  subcore_axis_name="subcore")

    @pl.kernel(out_type=jax.ShapeDtypeStruct((_B, _D), jnp.float32),
               mesh=mesh)
    def gather_kernel(f_hbm, i_hbm, o_hbm):
        def body(i_vmem, o_vmem):
            pltpu.sync_copy(f_hbm.at[i_vmem.at[0]], o_vmem)

        pltpu.emit_pipeline(
            body,
            grid=(_B // _GW,),
            in_specs=[pl.BlockSpec((1, _GW), index_map=lambda i: (0, i))],
            out_specs=[pl.BlockSpec((_GW, _D), index_map=lambda i: (i, 0))],
            core_axis_name=("core", "subcore"),
            dimension_semantics=(pltpu.PARALLEL,),
        )(i_hbm, o_hbm)

    return gather_kernel(features, t2d)


def kernel(inputs, targets, momentum, features):
    del momentum
    t2d = targets.astype(jnp.int32).reshape(1, _B)
    gathered = _sc_gather(features, t2d)

    s, xn = pl.pallas_call(
        _stream_body,
        grid=(_N // _BLK,),
        in_specs=[
            pl.BlockSpec((_B, _D), lambda k: (0, 0)),
            pl.BlockSpec((_BLK, _D), lambda k: (k, 0)),
        ],
        out_specs=[
            pl.BlockSpec((_B, 1), lambda k: (0, 0)),
            pl.BlockSpec((_B, _D), lambda k: (0, 0)),
        ],
        out_shape=[
            jax.ShapeDtypeStruct((_B, 1), jnp.float32),
            jax.ShapeDtypeStruct((_B, _D), jnp.float32),
        ],
        scratch_shapes=[
            pltpu.VMEM((_B, _D), jnp.bfloat16),
            pltpu.VMEM((_B, 1), jnp.float32),
        ],
    )(inputs, features)

    loss = pl.pallas_call(
        _combine_body,
        out_shape=jax.ShapeDtypeStruct((1, 1), jnp.float32),
    )(s, xn, gathered)
    return loss[0, 0]
